# initial kernel scaffold (unmeasured)
import jax
import jax.numpy as jnp
from jax import lax
from jax.experimental import pallas as pl
from jax.experimental.pallas import tpu as pltpu

N_DEV = 32


def kernel(x, w_mat):
    m_per, k = x.shape
    _, n_per = w_mat.shape

    def body(x_ref, w_ref, out_ref, comm_ref, send_sems, recv_sems):
        my = lax.axis_index("i")
        left = lax.rem(my - 1 + N_DEV, N_DEV)
        right = lax.rem(my + 1, N_DEV)

        barrier_sem = pltpu.get_barrier_semaphore()
        for nbr in (left, right):
            pl.semaphore_signal(
                barrier_sem, inc=1,
                device_id=(nbr,), device_id_type=pl.DeviceIdType.MESH,
            )
        pl.semaphore_wait(barrier_sem, 2)

        w = w_ref[...].astype(jnp.bfloat16)

        def gemm_store(chunk, origin):
            y = jnp.dot(chunk, w, preferred_element_type=jnp.float32)
            y = jax.nn.gelu(y, approximate=True)
            out_ref[pl.ds(origin * m_per, m_per), :] = y

        comm_ref[0] = x_ref[...].astype(jnp.bfloat16)
        gemm_store(comm_ref[0], my)

        for h in range(N_DEV - 1):
            rdma = pltpu.make_async_remote_copy(
                src_ref=comm_ref.at[h],
                dst_ref=comm_ref.at[h + 1],
                send_sem=send_sems.at[h],
                recv_sem=recv_sems.at[h],
                device_id=(right,),
                device_id_type=pl.DeviceIdType.MESH,
            )
            rdma.start()
            rdma.wait()
            origin = lax.rem(my - h - 1 + N_DEV, N_DEV)
            gemm_store(comm_ref[h + 1], origin)

    return pl.pallas_call(
        body,
        out_shape=jax.ShapeDtypeStruct((N_DEV * m_per, n_per), jnp.float32),
        in_specs=[
            pl.BlockSpec(memory_space=pltpu.VMEM),
            pl.BlockSpec(memory_space=pltpu.VMEM),
        ],
        out_specs=pl.BlockSpec(memory_space=pltpu.VMEM),
        scratch_shapes=[
            pltpu.VMEM((N_DEV, m_per, k), jnp.bfloat16),
            pltpu.SemaphoreType.DMA((N_DEV - 1,)),
            pltpu.SemaphoreType.DMA((N_DEV - 1,)),
        ],
        compiler_params=pltpu.CompilerParams(collective_id=0),
    )(x, w_mat)


# baseline (device time: 444189 ns/iter reference)
import jax
import jax.numpy as jnp
from jax import lax
from jax.experimental import pallas as pl
from jax.experimental.pallas import tpu as pltpu

N_DEV = 32


def kernel(x, w_mat):
    m_per, k = x.shape
    _, n_per = w_mat.shape

    def body(x_ref, w_ref, out_ref, comm_ref, send_sems, recv_sems):
        my = lax.axis_index("i")
        left = lax.rem(my - 1 + N_DEV, N_DEV)
        right = lax.rem(my + 1, N_DEV)

        barrier_sem = pltpu.get_barrier_semaphore()
        for nbr in (left, right):
            pl.semaphore_signal(
                barrier_sem, inc=1,
                device_id=(nbr,), device_id_type=pl.DeviceIdType.MESH,
            )
        pl.semaphore_wait(barrier_sem, 2)

        w = w_ref[...].astype(jnp.bfloat16)

        def gemm_store(chunk, origin):
            y = jnp.dot(chunk, w, preferred_element_type=jnp.float32)
            y = jax.nn.gelu(y, approximate=True)
            out_ref[pl.ds(origin * m_per, m_per), :] = y

        comm_ref[0] = x_ref[...].astype(jnp.bfloat16)
        gemm_store(comm_ref[0], my)

        for h in range(N_DEV - 1):
            rdma = pltpu.make_async_remote_copy(
                src_ref=comm_ref.at[h],
                dst_ref=comm_ref.at[h + 1],
                send_sem=send_sems.at[h],
                recv_sem=recv_sems.at[h],
                device_id=(right,),
                device_id_type=pl.DeviceIdType.MESH,
            )
            rdma.start()
            rdma.wait()
            origin = lax.rem(my - h - 1 + N_DEV, N_DEV)
            gemm_store(comm_ref[h + 1], origin)

    return pl.pallas_call(
        body,
        out_shape=jax.ShapeDtypeStruct((N_DEV * m_per, n_per), jnp.float32),
        in_specs=[
            pl.BlockSpec(memory_space=pltpu.VMEM),
            pl.BlockSpec(memory_space=pltpu.VMEM),
        ],
        out_specs=pl.BlockSpec(memory_space=pltpu.VMEM),
        scratch_shapes=[
            pltpu.VMEM((N_DEV, m_per, k), jnp.bfloat16),
            pltpu.SemaphoreType.DMA((N_DEV - 1,)),
            pltpu.SemaphoreType.DMA((N_DEV - 1,)),
        ],
        compiler_params=pltpu.CompilerParams(
            collective_id=0,
            vmem_limit_bytes=60 * 1024 * 1024,
        ),
    )(x, w_mat)


# device time: 377032 ns/iter; 1.1781x vs baseline; 1.1781x over previous
import jax
import jax.numpy as jnp
from jax import lax
from jax.experimental import pallas as pl
from jax.experimental.pallas import tpu as pltpu

N_DEV = 32
R_HOPS = N_DEV // 2
L_HOPS = N_DEV - 1 - R_HOPS


def kernel(x, w_mat):
    m_per, k = x.shape
    _, n_per = w_mat.shape

    def body(x_ref, w_ref, out_ref, comm_r, comm_l,
             send_r, recv_r, send_l, recv_l):
        my = lax.axis_index("i")
        left = lax.rem(my - 1 + N_DEV, N_DEV)
        right = lax.rem(my + 1, N_DEV)

        barrier_sem = pltpu.get_barrier_semaphore()
        for nbr in (left, right):
            pl.semaphore_signal(
                barrier_sem, inc=1,
                device_id=(nbr,), device_id_type=pl.DeviceIdType.MESH,
            )
        pl.semaphore_wait(barrier_sem, 2)

        comm_r[0] = x_ref[...].astype(jnp.bfloat16)

        def r_rdma(h):
            return pltpu.make_async_remote_copy(
                src_ref=comm_r.at[h],
                dst_ref=comm_r.at[h + 1],
                send_sem=send_r.at[h],
                recv_sem=recv_r.at[h],
                device_id=(right,),
                device_id_type=pl.DeviceIdType.MESH,
            )

        def l_rdma(h):
            return pltpu.make_async_remote_copy(
                src_ref=comm_r.at[0] if h == 0 else comm_l.at[h],
                dst_ref=comm_l.at[h + 1],
                send_sem=send_l.at[h],
                recv_sem=recv_l.at[h],
                device_id=(left,),
                device_id_type=pl.DeviceIdType.MESH,
            )

        w = w_ref[...].astype(jnp.bfloat16)

        def gemm_store(chunk, origin):
            y = jnp.dot(chunk, w, preferred_element_type=jnp.float32)
            y = jax.nn.gelu(y, approximate=True)
            out_ref[pl.ds(origin * m_per, m_per), :] = y

        first_r = r_rdma(0)
        first_r.start()
        first_l = l_rdma(0)
        first_l.start()
        gemm_store(comm_r[0], my)

        rdmas_r = [first_r] + [None] * (R_HOPS - 1)
        rdmas_l = [first_l] + [None] * (L_HOPS - 1)
        for h in range(R_HOPS):
            rdmas_r[h].wait()
            if h + 1 < R_HOPS:
                rdmas_r[h + 1] = r_rdma(h + 1)
                rdmas_r[h + 1].start()
            if h < L_HOPS:
                rdmas_l[h].wait()
                if h + 1 < L_HOPS:
                    rdmas_l[h + 1] = l_rdma(h + 1)
                    rdmas_l[h + 1].start()
            gemm_store(comm_r[h + 1], lax.rem(my - h - 1 + N_DEV, N_DEV))
            if h < L_HOPS:
                gemm_store(comm_l[h + 1], lax.rem(my + h + 1, N_DEV))

    return pl.pallas_call(
        body,
        out_shape=jax.ShapeDtypeStruct((N_DEV * m_per, n_per), jnp.float32),
        in_specs=[
            pl.BlockSpec(memory_space=pltpu.VMEM),
            pl.BlockSpec(memory_space=pltpu.VMEM),
        ],
        out_specs=pl.BlockSpec(memory_space=pltpu.VMEM),
        scratch_shapes=[
            pltpu.VMEM((R_HOPS + 1, m_per, k), jnp.bfloat16),
            pltpu.VMEM((L_HOPS + 1, m_per, k), jnp.bfloat16),
            pltpu.SemaphoreType.DMA((R_HOPS,)),
            pltpu.SemaphoreType.DMA((R_HOPS,)),
            pltpu.SemaphoreType.DMA((L_HOPS,)),
            pltpu.SemaphoreType.DMA((L_HOPS,)),
        ],
        compiler_params=pltpu.CompilerParams(
            collective_id=0,
            vmem_limit_bytes=60 * 1024 * 1024,
        ),
    )(x, w_mat)


# device time: 195796 ns/iter; 2.2686x vs baseline; 1.9256x over previous
import jax
import jax.numpy as jnp
from jax import lax
from jax.experimental import pallas as pl
from jax.experimental.pallas import tpu as pltpu

N_DEV = 32
R_HOPS = N_DEV // 2
L_HOPS = N_DEV - 1 - R_HOPS
SUB = 2

_PLANE = [(0, 0), (1, 0), (1, 1), (0, 1), (0, 2), (1, 2), (1, 3), (0, 3)]
_COORD_OF_LOGICAL = [(x, y, z) for z in range(4) for (x, y) in _PLANE]

_H = [(0, 0), (1, 0), (2, 0), (3, 0), (3, 1), (2, 1), (1, 1), (1, 2),
      (2, 2), (3, 2), (3, 3), (2, 3), (1, 3), (0, 3), (0, 2), (0, 1)]
_RING_COORDS = [(0, y, z) for (y, z) in _H] + [(1, y, z) for (y, z) in reversed(_H)]

_LOGICAL_OF_COORD = {c: l for l, c in enumerate(_COORD_OF_LOGICAL)}
ID_AT_POS = [_LOGICAL_OF_COORD[c] for c in _RING_COORDS]
POS_OF_ID = [0] * N_DEV
for _p, _l in enumerate(ID_AT_POS):
    POS_OF_ID[_l] = _p


def kernel(x, w_mat):
    m_per, k = x.shape
    _, n_per = w_mat.shape
    sub_m = m_per // SUB

    def body(pos_tab, id_tab, x_ref, w_ref, out_ref, comm_r, comm_l,
             send_r, recv_r, send_l, recv_l):
        my = lax.axis_index("i")
        pos = pos_tab[my]

        def id_at(expr):
            return id_tab[lax.rem(expr + 2 * N_DEV, N_DEV)]

        left = id_at(pos - 1)
        right = id_at(pos + 1)

        barrier_sem = pltpu.get_barrier_semaphore()
        for nbr in (left, right):
            pl.semaphore_signal(
                barrier_sem, inc=1,
                device_id=(nbr,), device_id_type=pl.DeviceIdType.MESH,
            )
        pl.semaphore_wait(barrier_sem, 2)

        comm_r[0] = x_ref[...].astype(jnp.bfloat16)

        def sub_slice(ref, h, s):
            return ref.at[h, pl.ds(s * sub_m, sub_m), :]

        def r_rdma(h, s):
            return pltpu.make_async_remote_copy(
                src_ref=sub_slice(comm_r, h, s),
                dst_ref=sub_slice(comm_r, h + 1, s),
                send_sem=send_r.at[h, s],
                recv_sem=recv_r.at[h, s],
                device_id=(right,),
                device_id_type=pl.DeviceIdType.MESH,
            )

        def l_rdma(h, s):
            return pltpu.make_async_remote_copy(
                src_ref=sub_slice(comm_r if h == 0 else comm_l, 0 if h == 0 else h, s),
                dst_ref=sub_slice(comm_l, h + 1, s),
                send_sem=send_l.at[h, s],
                recv_sem=recv_l.at[h, s],
                device_id=(left,),
                device_id_type=pl.DeviceIdType.MESH,
            )

        w = w_ref[...].astype(jnp.bfloat16)

        def gemm_store(chunk, origin):
            y = jnp.dot(chunk, w, preferred_element_type=jnp.float32)
            y = jax.nn.gelu(y, approximate=True)
            out_ref[pl.ds(origin * m_per, m_per), :] = y

        rr = {(0, s): r_rdma(0, s) for s in range(SUB)}
        ll = {(0, s): l_rdma(0, s) for s in range(SUB)}
        for s in range(SUB):
            rr[0, s].start()
            ll[0, s].start()
        gemm_store(comm_r[0], my)

        for h in range(R_HOPS):
            has_l = h < L_HOPS
            for s in range(SUB):
                rr[h, s].wait()
                if h + 1 < R_HOPS:
                    rr[h + 1, s] = r_rdma(h + 1, s)
                    rr[h + 1, s].start()
                if has_l:
                    ll[h, s].wait()
                    if h + 1 < L_HOPS:
                        ll[h + 1, s] = l_rdma(h + 1, s)
                        ll[h + 1, s].start()
            gemm_store(comm_r[h + 1], id_at(pos - h - 1))
            if has_l:
                gemm_store(comm_l[h + 1], id_at(pos + h + 1))

    pos_tab = jnp.asarray(POS_OF_ID, dtype=jnp.int32)
    id_tab = jnp.asarray(ID_AT_POS, dtype=jnp.int32)

    return pl.pallas_call(
        body,
        out_shape=jax.ShapeDtypeStruct((N_DEV * m_per, n_per), jnp.float32),
        in_specs=[
            pl.BlockSpec(memory_space=pltpu.SMEM),
            pl.BlockSpec(memory_space=pltpu.SMEM),
            pl.BlockSpec(memory_space=pltpu.VMEM),
            pl.BlockSpec(memory_space=pltpu.VMEM),
        ],
        out_specs=pl.BlockSpec(memory_space=pltpu.VMEM),
        scratch_shapes=[
            pltpu.VMEM((R_HOPS + 1, m_per, k), jnp.bfloat16),
            pltpu.VMEM((L_HOPS + 1, m_per, k), jnp.bfloat16),
            pltpu.SemaphoreType.DMA((R_HOPS, SUB)),
            pltpu.SemaphoreType.DMA((R_HOPS, SUB)),
            pltpu.SemaphoreType.DMA((L_HOPS, SUB)),
            pltpu.SemaphoreType.DMA((L_HOPS, SUB)),
        ],
        compiler_params=pltpu.CompilerParams(
            collective_id=0,
            vmem_limit_bytes=60 * 1024 * 1024,
        ),
    )(pos_tab, id_tab, x, w_mat)
